# baseline (speedup 1.0000x reference)
import functools

import jax
import jax.numpy as jnp
from jax import lax
from jax.experimental import pallas as pl
from jax.experimental.pallas import tpu as pltpu
from jax.experimental.pallas import tpu_sc as plsc

_B = 128
_D = 1024
_C = 1000
_K = 10
_SCALE = 20.0
_CP = 1024
_CB = 200
_NP = 1536
_PER = 48
_NCHUNK = _NP // 16

_HI = jax.lax.Precision.HIGHEST


def _col_stats(x, nr_sentinel):
    nr, nc = x.shape
    m = jnp.max(x, axis=0, keepdims=True)
    e = jnp.exp(x - m)
    s = jnp.sum(e, axis=0, keepdims=True)
    sx = jnp.sum(e * x, axis=0, keepdims=True)
    ent = m + jnp.log(s) - sx / s
    conf = 1.0 / s
    row = jax.lax.broadcasted_iota(jnp.int32, (nr, nc), 0)
    idx = jnp.min(jnp.where(x == m, row, nr_sentinel), axis=0, keepdims=True)
    return idx.astype(jnp.float32), ent, conf


def _norm_rows(x):
    n = jnp.sqrt(jnp.sum(x * x, axis=1, keepdims=True))
    return x / jnp.maximum(n, 1e-12)


def _stats_kernel(feat_ref, raw_t_ref, aug_t_ref, wp_ref, bc_ref,
                  y_ref, k_ref, w_ref):
    b_col = bc_ref[...]
    feat = feat_ref[...]
    W = wp_ref[0:_C, :]

    lt = jax.lax.dot_general(W, feat, (((1,), (1,)), ((), ())),
                             preferred_element_type=jnp.float32) + b_col
    y_f, ent_f, conf_f = _col_stats(lt, _C)
    rmax = jnp.max(raw_t_ref[...], axis=0, keepdims=True)
    amax = jnp.max(aug_t_ref[...], axis=0, keepdims=True)
    rows = jax.lax.broadcasted_iota(jnp.int32, (_C, _B), 0)
    r_idx = jnp.min(jnp.where(raw_t_ref[...] == rmax, rows, _C), axis=0,
                    keepdims=True)
    a_idx = jnp.min(jnp.where(aug_t_ref[...] == amax, rows, _C), axis=0,
                    keepdims=True)
    maskv = (r_idx == a_idx).astype(jnp.float32)
    am = jnp.max(maskv)
    y_f_mod = maskv * y_f + (1.0 - maskv) * float(_C)
    y_ref[8:9, :] = y_f_mod
    k_ref[8:9, :] = y_f_mod * 1000.0 + maskv * ent_f
    w_ref[8:9, :] = jnp.maximum(conf_f, 1e-6) * maskv
    pad3 = jnp.zeros((3, _B), jnp.float32)
    y_ref[9:12, :] = pad3 - 3.0
    k_ref[9:12, :] = pad3 - 8000.0
    w_ref[9:12, :] = pad3
    w_ref[9:10, :] = jnp.full((1, _B), am)

    for si in range(8):
        blk = wp_ref[si * _B:(si + 1) * _B, :]
        t = jax.lax.dot_general(W, blk, (((1,), (1,)), ((), ())),
                                preferred_element_type=jnp.float32) + b_col
        y_w, ent_w, conf_w = _col_stats(t, _C)
        real = (jax.lax.broadcasted_iota(jnp.int32, (1, _B), 1)
                + si * _B < _C).astype(jnp.float32)
        y_ref[si:si + 1, :] = real * y_w + (1.0 - real) * (-2.0)
        k_ref[si:si + 1, :] = (real * (y_w * 1000.0 + ent_w)
                               + (1.0 - real) * (-5000.0))
        w_ref[si:si + 1, :] = real * jnp.maximum(conf_w, 1e-6)


def _sc_rank_kernel(k_hbm, kept_hbm, k_v, kept_v):
    wid = lax.axis_index("s") * 2 + lax.axis_index("c")
    base = wid * _PER
    pltpu.sync_copy(k_hbm, k_v)
    lane = lax.iota(jnp.int32, 16).astype(jnp.float32)

    for ic in range(_PER // 16):
        k_mine = k_v[pl.ds(base + ic * 16, 16)]
        k_lo = k_mine - 900.0
        idx_mine = lane + (base + ic * 16).astype(jnp.float32)

        def body(jc, rank, _k=k_mine, _lo=k_lo, _idx=idx_mine):
            k_jc = k_v[pl.ds(jc * 16, 16)]
            for l in range(16):
                k_j = k_jc[l]
                idx_j = (jc * 16 + l).astype(jnp.float32)
                beat = (k_j < _k) | ((k_j == _k) & (idx_j < _idx))
                rank = rank + jnp.where((k_j > _lo) & beat, 1.0, 0.0)
            return rank

        rank = lax.fori_loop(0, _NCHUNK, body,
                             jnp.zeros((16,), jnp.float32))
        kept_v[pl.ds(ic * 16, 16)] = jnp.where(rank < float(_K), 1.0, 0.0)

    pltpu.sync_copy(kept_v, kept_hbm.at[pl.ds(base, _PER)])


_sc_rank = functools.partial(
    pl.kernel,
    mesh=plsc.VectorSubcoreMesh(core_axis_name="c", subcore_axis_name="s"),
    out_type=jax.ShapeDtypeStruct((_NP,), jnp.float32),
    scratch_types=[
        pltpu.VMEM((_NP,), jnp.float32),
        pltpu.VMEM((_PER,), jnp.float32),
    ],
)(_sc_rank_kernel)


def _cent_kernel(feat_ref, wp_ref, y_ref, w_ref, kept_ref, out_ref,
                 wn_ref, fn_ref, cent_ref):
    fn_ref[...] = _norm_rows(feat_ref[...])
    for si in range(8):
        wn_ref[si * _B:(si + 1) * _B, :] = _norm_rows(
            wp_ref[si * _B:(si + 1) * _B, :])

    am = w_ref[9:10, :]
    for ci in range(5):
        cls = (jax.lax.broadcasted_iota(jnp.int32, (_CB, 1), 0)
               + ci * _CB).astype(jnp.float32)
        w_f = w_ref[8:9, :] * am * kept_ref[8:9, :]
        a_f = jnp.where(y_ref[8:9, :] == cls, w_f, 0.0)
        acc = jax.lax.dot_general(a_f, fn_ref[...], (((1,), (0,)), ((), ())),
                                  precision=_HI,
                                  preferred_element_type=jnp.float32)
        for sj in range(8):
            w_w = w_ref[sj:sj + 1, :] * (am * kept_ref[sj:sj + 1, :]
                                         + (1.0 - am))
            a_w = jnp.where(y_ref[sj:sj + 1, :] == cls, w_w, 0.0)
            acc += jax.lax.dot_general(
                a_w, wn_ref[sj * _B:(sj + 1) * _B, :], (((1,), (0,)), ((), ())),
                precision=_HI, preferred_element_type=jnp.float32)
        cent_ref[ci * _CB:(ci + 1) * _CB, :] = _norm_rows(acc)

    out_ref[...] = _SCALE * jax.lax.dot_general(
        fn_ref[...], cent_ref[...], (((1,), (1,)), ((), ())),
        precision=_HI, preferred_element_type=jnp.float32)


def kernel(feat, logits_raw, logits_aug, W, b):
    w_pad = jnp.pad(W, ((0, _CP - _C), (0, 0)))
    b_col = b.reshape(_C, 1)

    y12, k12, w12 = pl.pallas_call(
        _stats_kernel,
        out_shape=[jax.ShapeDtypeStruct((12, _B), jnp.float32)] * 3,
        compiler_params=pltpu.CompilerParams(
            vmem_limit_bytes=100 * 1024 * 1024,
        ),
    )(feat, logits_raw.T, logits_aug.T, w_pad, b_col)

    kept = _sc_rank(k12.reshape(_NP))
    kept12 = kept.reshape(12, _B)

    return pl.pallas_call(
        _cent_kernel,
        out_shape=jax.ShapeDtypeStruct((_B, _C), jnp.float32),
        scratch_shapes=[
            pltpu.VMEM((_CP, _D), jnp.float32),
            pltpu.VMEM((_B, _D), jnp.float32),
            pltpu.VMEM((_C, _D), jnp.float32),
        ],
        compiler_params=pltpu.CompilerParams(
            vmem_limit_bytes=100 * 1024 * 1024,
        ),
    )(feat, w_pad, y12, w12, kept12)


# speedup vs baseline: 1.0761x; 1.0761x over previous
import functools

import jax
import jax.numpy as jnp
from jax import lax
from jax.experimental import pallas as pl
from jax.experimental.pallas import tpu as pltpu
from jax.experimental.pallas import tpu_sc as plsc

_B = 128
_D = 1024
_C = 1000
_K = 10
_SCALE = 20.0
_CP = 1024
_CB = 200
_NP = 1536
_PER = 48
_NCHUNK = 1152 // 16

_HI = jax.lax.Precision.HIGHEST


def _col_stats(x, nr_sentinel):
    nr, nc = x.shape
    m = jnp.max(x, axis=0, keepdims=True)
    e = jnp.exp(x - m)
    s = jnp.sum(e, axis=0, keepdims=True)
    sx = jnp.sum(e * x, axis=0, keepdims=True)
    ent = m + jnp.log(s) - sx / s
    conf = 1.0 / s
    row = jax.lax.broadcasted_iota(jnp.int32, (nr, nc), 0)
    idx = jnp.min(jnp.where(x == m, row, nr_sentinel), axis=0, keepdims=True)
    return idx.astype(jnp.float32), ent, conf


def _norm_rows(x):
    n = jnp.sqrt(jnp.sum(x * x, axis=1, keepdims=True))
    return x / jnp.maximum(n, 1e-12)


def _stats_kernel(feat_ref, raw_t_ref, aug_t_ref, wp_ref, bc_ref,
                  y_ref, k_ref, w_ref):
    b_col = bc_ref[...]
    feat = feat_ref[...]
    W = wp_ref[0:_C, :]

    lt = jax.lax.dot_general(W, feat, (((1,), (1,)), ((), ())),
                             preferred_element_type=jnp.float32) + b_col
    y_f, ent_f, conf_f = _col_stats(lt, _C)
    rmax = jnp.max(raw_t_ref[...], axis=0, keepdims=True)
    amax = jnp.max(aug_t_ref[...], axis=0, keepdims=True)
    rows = jax.lax.broadcasted_iota(jnp.int32, (_C, _B), 0)
    r_idx = jnp.min(jnp.where(raw_t_ref[...] == rmax, rows, _C), axis=0,
                    keepdims=True)
    a_idx = jnp.min(jnp.where(aug_t_ref[...] == amax, rows, _C), axis=0,
                    keepdims=True)
    maskv = (r_idx == a_idx).astype(jnp.float32)
    am = jnp.max(maskv)
    y_f_mod = maskv * y_f + (1.0 - maskv) * float(_C)
    y_ref[8:9, :] = y_f_mod
    k_ref[8:9, :] = y_f_mod * 1000.0 + maskv * ent_f
    w_ref[8:9, :] = jnp.maximum(conf_f, 1e-6) * maskv
    pad3 = jnp.zeros((3, _B), jnp.float32)
    y_ref[9:12, :] = pad3 - 3.0
    k_ref[9:12, :] = pad3 - 8000.0
    w_ref[9:12, :] = pad3
    w_ref[9:10, :] = jnp.full((1, _B), am)

    for si in range(8):
        blk = wp_ref[si * _B:(si + 1) * _B, :]
        t = jax.lax.dot_general(W, blk, (((1,), (1,)), ((), ())),
                                preferred_element_type=jnp.float32) + b_col
        y_w, ent_w, conf_w = _col_stats(t, _C)
        real = (jax.lax.broadcasted_iota(jnp.int32, (1, _B), 1)
                + si * _B < _C).astype(jnp.float32)
        y_ref[si:si + 1, :] = real * y_w + (1.0 - real) * (-2.0)
        k_ref[si:si + 1, :] = (real * (y_w * 1000.0 + ent_w)
                               + (1.0 - real) * (-5000.0))
        w_ref[si:si + 1, :] = real * jnp.maximum(conf_w, 1e-6)


def _sc_rank_kernel(k_hbm, kept_hbm, k_v, kept_v):
    wid = lax.axis_index("s") * 2 + lax.axis_index("c")
    base = wid * _PER
    pltpu.sync_copy(k_hbm, k_v)
    lane = lax.iota(jnp.int32, 16).astype(jnp.float32)

    for ic in range(_PER // 16):
        k_mine = k_v[pl.ds(base + ic * 16, 16)]
        k_lo = k_mine - 900.0
        idx_mine = lane + (base + ic * 16).astype(jnp.float32)

        def body(jc, rank, _k=k_mine, _lo=k_lo, _idx=idx_mine):
            k_jc = k_v[pl.ds(jc * 16, 16)]
            for l in range(16):
                k_j = k_jc[l]
                idx_j = (jc * 16 + l).astype(jnp.float32)
                beat = (k_j < _k) | ((k_j == _k) & (idx_j < _idx))
                rank = rank + jnp.where((k_j > _lo) & beat, 1.0, 0.0)
            return rank

        rank = lax.fori_loop(0, _NCHUNK, body,
                             jnp.zeros((16,), jnp.float32))
        kept_v[pl.ds(ic * 16, 16)] = jnp.where(rank < float(_K), 1.0, 0.0)

    pltpu.sync_copy(kept_v, kept_hbm.at[pl.ds(base, _PER)])


_sc_rank = functools.partial(
    pl.kernel,
    mesh=plsc.VectorSubcoreMesh(core_axis_name="c", subcore_axis_name="s"),
    out_type=jax.ShapeDtypeStruct((_NP,), jnp.float32),
    scratch_types=[
        pltpu.VMEM((_NP,), jnp.float32),
        pltpu.VMEM((_PER,), jnp.float32),
    ],
)(_sc_rank_kernel)


def _cent_kernel(feat_ref, wp_ref, y_ref, w_ref, kept_ref, out_ref,
                 wn_ref, fn_ref, cent_ref):
    fn_ref[...] = _norm_rows(feat_ref[...])
    for si in range(8):
        wn_ref[si * _B:(si + 1) * _B, :] = _norm_rows(
            wp_ref[si * _B:(si + 1) * _B, :])

    am = w_ref[9:10, :]
    for ci in range(5):
        cls = (jax.lax.broadcasted_iota(jnp.int32, (_CB, 1), 0)
               + ci * _CB).astype(jnp.float32)
        w_f = w_ref[8:9, :] * am * kept_ref[8:9, :]
        a_f = jnp.where(y_ref[8:9, :] == cls, w_f, 0.0)
        acc = jax.lax.dot_general(a_f, fn_ref[...], (((1,), (0,)), ((), ())),
                                  precision=_HI,
                                  preferred_element_type=jnp.float32)
        for sj in range(8):
            w_w = w_ref[sj:sj + 1, :] * (am * kept_ref[sj:sj + 1, :]
                                         + (1.0 - am))
            a_w = jnp.where(y_ref[sj:sj + 1, :] == cls, w_w, 0.0)
            acc += jax.lax.dot_general(
                a_w, wn_ref[sj * _B:(sj + 1) * _B, :], (((1,), (0,)), ((), ())),
                precision=_HI, preferred_element_type=jnp.float32)
        cent_ref[ci * _CB:(ci + 1) * _CB, :] = _norm_rows(acc)

    out_ref[...] = _SCALE * jax.lax.dot_general(
        fn_ref[...], cent_ref[...], (((1,), (1,)), ((), ())),
        precision=_HI, preferred_element_type=jnp.float32)


def kernel(feat, logits_raw, logits_aug, W, b):
    w_pad = jnp.pad(W, ((0, _CP - _C), (0, 0)))
    b_col = b.reshape(_C, 1)

    y12, k12, w12 = pl.pallas_call(
        _stats_kernel,
        out_shape=[jax.ShapeDtypeStruct((12, _B), jnp.float32)] * 3,
        compiler_params=pltpu.CompilerParams(
            vmem_limit_bytes=100 * 1024 * 1024,
        ),
    )(feat, logits_raw.T, logits_aug.T, w_pad, b_col)

    kept = _sc_rank(k12.reshape(_NP))
    kept12 = kept.reshape(12, _B)

    return pl.pallas_call(
        _cent_kernel,
        out_shape=jax.ShapeDtypeStruct((_B, _C), jnp.float32),
        scratch_shapes=[
            pltpu.VMEM((_CP, _D), jnp.float32),
            pltpu.VMEM((_B, _D), jnp.float32),
            pltpu.VMEM((_C, _D), jnp.float32),
        ],
        compiler_params=pltpu.CompilerParams(
            vmem_limit_bytes=100 * 1024 * 1024,
        ),
    )(feat, w_pad, y12, w12, kept12)


# speedup vs baseline: 1.6099x; 1.4961x over previous
import jax
import jax.numpy as jnp
from jax.experimental import pallas as pl
from jax.experimental.pallas import tpu as pltpu

_B = 128
_D = 1024
_C = 1000
_K = 10
_SCALE = 20.0
_CP = 1024
_CB = 200

_HI = jax.lax.Precision.HIGHEST


def _col_stats(x, n_valid):
    nr, nc = x.shape
    m = jnp.max(x, axis=0, keepdims=True)
    e = jnp.exp(x - m)
    s = jnp.sum(e, axis=0, keepdims=True)
    sx = jnp.sum(e * x, axis=0, keepdims=True)
    ent = m + jnp.log(s) - sx / s
    conf = 1.0 / s
    row = jax.lax.broadcasted_iota(jnp.int32, (nr, nc), 0)
    idx = jnp.min(jnp.where(x == m, row, nr), axis=0, keepdims=True)
    return idx.astype(jnp.float32), ent, conf


def _norm_rows(x):
    n = jnp.sqrt(jnp.sum(x * x, axis=1, keepdims=True))
    return x / jnp.maximum(n, 1e-12)


def _fused_kernel(feat_ref, raw_t_ref, aug_t_ref, wp_ref, bc_ref, out_ref,
                  wn_ref, fn_ref, cent_ref,
                  y_w_ref, k_w_ref, w_w_ref,
                  y_f_ref, k_f_ref, w_f_ref, am_ref):
    b_col = bc_ref[...]
    feat = feat_ref[...]
    W = wp_ref[0:_C, :]

    fn_ref[...] = _norm_rows(feat)
    lt = jax.lax.dot_general(W, feat, (((1,), (1,)), ((), ())),
                             preferred_element_type=jnp.float32) + b_col
    y_f, ent_f, conf_f = _col_stats(lt, _C)
    rmax = jnp.max(raw_t_ref[...], axis=0, keepdims=True)
    amax = jnp.max(aug_t_ref[...], axis=0, keepdims=True)
    rows = jax.lax.broadcasted_iota(jnp.int32, (_C, _B), 0)
    r_idx = jnp.min(jnp.where(raw_t_ref[...] == rmax, rows, _C), axis=0,
                    keepdims=True)
    a_idx = jnp.min(jnp.where(aug_t_ref[...] == amax, rows, _C), axis=0,
                    keepdims=True)
    maskv = (r_idx == a_idx).astype(jnp.float32)
    am_ref[0, 0] = jnp.max(maskv)
    y_f_mod = maskv * y_f + (1.0 - maskv) * float(_C)
    key_f = y_f_mod * 1000.0 + maskv * ent_f
    y_f_ref[0:1, :] = y_f_mod
    k_f_ref[0:1, :] = key_f
    w_f_ref[0:1, :] = jnp.maximum(conf_f, 1e-6) * maskv
    pad = jnp.zeros((7, _B), jnp.float32)
    y_f_ref[1:8, :] = pad - 1.0
    k_f_ref[1:8, :] = pad
    w_f_ref[1:8, :] = pad

    for si in range(8):
        blk = wp_ref[si * _B:(si + 1) * _B, :]
        t = jax.lax.dot_general(W, blk, (((1,), (1,)), ((), ())),
                                preferred_element_type=jnp.float32) + b_col
        y_w, ent_w, conf_w = _col_stats(t, _C)
        real = (jax.lax.broadcasted_iota(jnp.int32, (1, _B), 1)
                + si * _B < _C).astype(jnp.float32)
        y_w_ref[si:si + 1, :] = real * y_w + (1.0 - real) * (-2.0)
        k_w_ref[si:si + 1, :] = real * (y_w * 1000.0 + ent_w)
        w_w_ref[si:si + 1, :] = real * jnp.maximum(conf_w, 1e-6)
        wn_ref[si * _B:(si + 1) * _B, :] = _norm_rows(blk)

    am = am_ref[0, 0]

    kt_w = jnp.transpose(k_w_ref[...])
    yt_w = jnp.transpose(y_w_ref[...])
    kt_f = jnp.transpose(k_f_ref[...])
    yt_f = jnp.transpose(y_f_ref[...])
    lane_col = jax.lax.broadcasted_iota(jnp.int32, (_B, 1), 0)
    lane_row = jax.lax.broadcasted_iota(jnp.int32, (1, _B), 1)

    for si in range(8):
        key_i = k_w_ref[si:si + 1, :]
        y_i = y_w_ref[si:si + 1, :]
        idx_i = lane_row + si * _B
        rank = jnp.zeros((1, _B), jnp.float32)
        for sj in range(8):
            key_j = kt_w[:, sj:sj + 1]
            y_j = yt_w[:, sj:sj + 1]
            idx_j = lane_col + sj * _B
            lt = (key_j < key_i) | ((key_j == key_i) & (idx_j < idx_i))
            rank += jnp.sum(jnp.where((y_j == y_i) & lt, 1.0, 0.0),
                            axis=0, keepdims=True)
        lt = kt_f[:, 0:1] < key_i
        rank += jnp.sum(jnp.where((yt_f[:, 0:1] == y_i) & lt, 1.0, 0.0),
                        axis=0, keepdims=True)
        kept = (rank < float(_K)).astype(jnp.float32)
        kept = am * kept + (1.0 - am)
        w_w_ref[si:si + 1, :] = w_w_ref[si:si + 1, :] * kept

    key_i = k_f_ref[0:1, :]
    y_i = y_f_ref[0:1, :]
    rank = jnp.zeros((1, _B), jnp.float32)
    for sj in range(8):
        lt = kt_w[:, sj:sj + 1] <= key_i
        rank += jnp.sum(jnp.where((yt_w[:, sj:sj + 1] == y_i) & lt, 1.0, 0.0),
                        axis=0, keepdims=True)
    lt = ((kt_f[:, 0:1] < key_i)
          | ((kt_f[:, 0:1] == key_i) & (lane_col < lane_row)))
    rank += jnp.sum(jnp.where((yt_f[:, 0:1] == y_i) & lt, 1.0, 0.0),
                    axis=0, keepdims=True)
    kept_f = am * (rank < float(_K)).astype(jnp.float32)
    w_f_ref[0:1, :] = w_f_ref[0:1, :] * kept_f

    for ci in range(5):
        cls = (jax.lax.broadcasted_iota(jnp.int32, (_CB, 1), 0)
               + ci * _CB).astype(jnp.float32)
        a_f = jnp.where(y_f_ref[0:1, :] == cls, w_f_ref[0:1, :], 0.0)
        acc = jax.lax.dot_general(a_f, fn_ref[...], (((1,), (0,)), ((), ())),
                                  precision=_HI,
                                  preferred_element_type=jnp.float32)
        for sj in range(8):
            a_w = jnp.where(y_w_ref[sj:sj + 1, :] == cls,
                            w_w_ref[sj:sj + 1, :], 0.0)
            acc += jax.lax.dot_general(
                a_w, wn_ref[sj * _B:(sj + 1) * _B, :], (((1,), (0,)), ((), ())),
                precision=_HI, preferred_element_type=jnp.float32)
        cent_ref[ci * _CB:(ci + 1) * _CB, :] = _norm_rows(acc)

    out_ref[...] = _SCALE * jax.lax.dot_general(
        fn_ref[...], cent_ref[...], (((1,), (1,)), ((), ())),
        precision=_HI, preferred_element_type=jnp.float32)


def kernel(feat, logits_raw, logits_aug, W, b):
    w_pad = jnp.pad(W, ((0, _CP - _C), (0, 0)))
    b_col = b.reshape(_C, 1)
    raw_t = logits_raw.T
    aug_t = logits_aug.T
    return pl.pallas_call(
        _fused_kernel,
        out_shape=jax.ShapeDtypeStruct((_B, _C), jnp.float32),
        scratch_shapes=[
            pltpu.VMEM((_CP, _D), jnp.float32),
            pltpu.VMEM((_B, _D), jnp.float32),
            pltpu.VMEM((_C, _D), jnp.float32),
            pltpu.VMEM((8, _B), jnp.float32),
            pltpu.VMEM((8, _B), jnp.float32),
            pltpu.VMEM((8, _B), jnp.float32),
            pltpu.VMEM((8, _B), jnp.float32),
            pltpu.VMEM((8, _B), jnp.float32),
            pltpu.VMEM((8, _B), jnp.float32),
            pltpu.SMEM((1, 1), jnp.float32),
        ],
        compiler_params=pltpu.CompilerParams(
            vmem_limit_bytes=100 * 1024 * 1024,
        ),
    )(feat, raw_t, aug_t, w_pad, b_col)
